# baseline (speedup 1.0000x reference)
import functools

import jax
import jax.numpy as jnp
from jax import lax
from jax.experimental import pallas as pl
from jax.experimental.pallas import tpu as pltpu
from jax.experimental.pallas import tpu_sc as plsc

NC = 2
NS = 16
NW = NC * NS
D = 128
F32 = jnp.float32


@functools.lru_cache(maxsize=None)
def _make_gather(rows, k, w=D):
    per = rows // NW
    nch = per // k
    assert per % k == 0 and per % 8 == 0 and k % 8 == 0
    mesh = plsc.VectorSubcoreMesh(core_axis_name="c", subcore_axis_name="s")

    def body(table_hbm, idx_hbm, out_hbm,
             idx0, idx1, buf0, buf1, s0, s1, w0, w1):
        wid = lax.axis_index("s") * NC + lax.axis_index("c")
        base = wid * per
        slots = ((idx0, buf0, s0, w0), (idx1, buf1, s1, w1))

        def load(slot, g):
            ib, rb, sem, _ = slots[slot]
            pltpu.sync_copy(idx_hbm.at[pl.ds(base + g * k, k)], ib)
            pltpu.async_copy(table_hbm.at[ib], rb, sem)

        def drain(slot, g):
            ib, rb, sem, wsem = slots[slot]
            pltpu.make_async_copy(table_hbm.at[ib], rb, sem).wait()
            pltpu.async_copy(rb, out_hbm.at[pl.ds(base + g * k, k)], wsem)

        def waitw(slot, g):
            _, rb, _, wsem = slots[slot]
            pltpu.make_async_copy(
                rb, out_hbm.at[pl.ds(base + g * k, k)], wsem).wait()

        load(0, 0)

        @pl.loop(0, nch, step=2)
        def _(g):
            @pl.when(g + 1 < nch)
            def _():
                @pl.when(g >= 1)
                def _():
                    waitw(1, g - 1)
                load(1, g + 1)
            drain(0, g)

            @pl.when(g + 2 < nch)
            def _():
                waitw(0, g)
                load(0, g + 2)

            @pl.when(g + 1 < nch)
            def _():
                drain(1, g + 1)

        waitw((nch - 1) % 2, nch - 1)
        if nch >= 2:
            waitw((nch - 2) % 2, nch - 2)

    def gather(table, idx):
        return pl.kernel(
            body,
            out_type=jax.ShapeDtypeStruct((rows, w), F32),
            mesh=mesh,
            scratch_types=[
                pltpu.VMEM((k,), jnp.int32),
                pltpu.VMEM((k,), jnp.int32),
                pltpu.VMEM((k, w), F32),
                pltpu.VMEM((k, w), F32),
                pltpu.SemaphoreType.DMA,
                pltpu.SemaphoreType.DMA,
                pltpu.SemaphoreType.DMA,
                pltpu.SemaphoreType.DMA,
            ],
        )(table, idx)

    return gather


@functools.lru_cache(maxsize=None)
def _make_scatter(E, Np, W, k, ones=False):
    per = E // NW
    nch = per // k
    rpt = Np // NS
    zrows = 80
    ka = 96
    kb = k - ka
    assert per % k == 0 and per % 8 == 0 and k % 8 == 0 and rpt % zrows == 0
    assert rpt % 8 == 0 and ka % 8 == 0 and kb % 8 == 0 and zrows <= ka
    mesh = plsc.VectorSubcoreMesh(core_axis_name="c", subcore_axis_name="s")

    def body(*args):
        if ones:
            dst_hbm, out_hbm, acc, idx0, idx1, row0 = args
        else:
            (msgs_hbm, dst_hbm, out_hbm, acc,
             idx0, idx1, row0, row1, s0, s1) = args
        cid = lax.axis_index("c")
        sid = lax.axis_index("s")
        wid = sid * NC + cid
        idxs = (idx0, idx1)

        @pl.loop(0, zrows)
        def _(i):
            for j in range(W // 16):
                row0.at[pl.ds(i, 1), pl.ds(j * 16, 16)][...] = jnp.zeros(
                    (1, 16), F32)

        for j in range(rpt // zrows):
            pltpu.sync_copy(row0.at[pl.ds(0, zrows)],
                            acc.at[pl.ds(sid * rpt + j * zrows, zrows)])
        plsc.subcore_barrier()

        if ones:
            @pl.loop(0, k)
            def _(i):
                for j in range(W // 16):
                    row0.at[pl.ds(i, 1), pl.ds(j * 16, 16)][...] = jnp.ones(
                        (1, 16), F32)

            pltpu.sync_copy(dst_hbm.at[pl.ds(wid * per, k)], idx0)

            @pl.loop(0, nch)
            def _(g):
                pltpu.sync_copy(row0, acc.at[idx0], add=True)

                @pl.when(g + 1 < nch)
                def _():
                    pltpu.sync_copy(
                        dst_hbm.at[pl.ds(wid * per + (g + 1) * k, k)], idx0)
        else:
            rows_ = (row0, row1)
            sems = (s0, s1)
            sizes = (ka, kb)

            def off_of(g, slot):
                return wid * per + g * k + slot * ka

            def load(slot, g):
                pltpu.sync_copy(
                    dst_hbm.at[pl.ds(off_of(g, slot), sizes[slot])],
                    idxs[slot])
                pltpu.async_copy(
                    msgs_hbm.at[pl.ds(off_of(g, slot), sizes[slot])],
                    rows_[slot], sems[slot])

            def addchunk(slot, g):
                pltpu.make_async_copy(
                    msgs_hbm.at[pl.ds(off_of(g, slot), sizes[slot])],
                    rows_[slot], sems[slot]).wait()
                pltpu.sync_copy(rows_[slot], acc.at[idxs[slot]], add=True)

            load(0, 0)

            @pl.loop(0, nch)
            def _(g):
                load(1, g)
                addchunk(0, g)

                @pl.when(g + 1 < nch)
                def _():
                    load(0, g + 1)
                addchunk(1, g)

        plsc.subcore_barrier()
        pltpu.sync_copy(acc.at[pl.ds(sid * rpt, rpt)],
                        out_hbm.at[cid, pl.ds(sid * rpt, rpt)])

    def scatter(*operands):
        if ones:
            scratch = [
                pltpu.VMEM_SHARED((Np, W), F32),
                pltpu.VMEM((k,), jnp.int32),
                pltpu.VMEM((k,), jnp.int32),
                pltpu.VMEM((k, W), F32),
            ]
        else:
            scratch = [
                pltpu.VMEM_SHARED((Np, W), F32),
                pltpu.VMEM((ka,), jnp.int32),
                pltpu.VMEM((kb,), jnp.int32),
                pltpu.VMEM((ka, W), F32),
                pltpu.VMEM((kb, W), F32),
                pltpu.SemaphoreType.DMA,
                pltpu.SemaphoreType.DMA,
            ]
        return pl.kernel(
            body,
            out_type=jax.ShapeDtypeStruct((NC, Np, W), F32),
            mesh=mesh,
            scratch_types=scratch,
        )(*operands)

    return scatter


def _tc_project(h, wab, bab, nb):
    n = h.shape[0]
    blk = n // nb

    def body(h_ref, w_ref, b_ref, o_ref):
        o_ref[0] = (
            jnp.dot(h_ref[...], w_ref[0], preferred_element_type=F32)
            + b_ref[0]
        )

    return pl.pallas_call(
        body,
        grid=(2, nb),
        in_specs=[
            pl.BlockSpec((blk, D), lambda i, j: (j, 0)),
            pl.BlockSpec((1, D, D), lambda i, j: (i, 0, 0)),
            pl.BlockSpec((1, 1, D), lambda i, j: (i, 0, 0)),
        ],
        out_specs=pl.BlockSpec((1, blk, D), lambda i, j: (i, j, 0)),
        out_shape=jax.ShapeDtypeStruct((2, n, D), F32),
    )(h, wab, bab)


def _tc_msg(G, attr, wc, w2, b2, eb):
    E = G.shape[0] // 2
    da = attr.shape[1]
    nb = E // eb

    def body(ga, gb, at, wc_ref, w2_ref, b2_ref, o_ref):
        x = ga[...] + gb[...] + jnp.dot(
            at[...], wc_ref[...], preferred_element_type=F32)
        hmid = jax.nn.silu(x)
        o_ref[...] = (
            jnp.dot(hmid, w2_ref[...], preferred_element_type=F32)
            + b2_ref[...]
        )

    return pl.pallas_call(
        body,
        grid=(nb,),
        in_specs=[
            pl.BlockSpec((eb, D), lambda i: (i, 0)),
            pl.BlockSpec((eb, D), lambda i, _nb=nb: (i + _nb, 0)),
            pl.BlockSpec((eb, da), lambda i: (i, 0)),
            pl.BlockSpec((da, D), lambda i: (0, 0)),
            pl.BlockSpec((D, D), lambda i: (0, 0)),
            pl.BlockSpec((1, D), lambda i: (0, 0)),
        ],
        out_specs=pl.BlockSpec((eb, D), lambda i: (i, 0)),
        out_shape=jax.ShapeDtypeStruct((E, D), F32),
    )(G, G, attr, wc, w2, b2)


def _tc_invcnt(C, nb):
    np_ = C.shape[1]
    blk = np_ // nb

    def body(c_ref, o_ref):
        o_ref[...] = 1.0 / jnp.maximum(c_ref[0] + c_ref[1], 1.0)

    return pl.pallas_call(
        body,
        grid=(nb,),
        in_specs=[pl.BlockSpec((NC, blk, D), lambda i: (0, i, 0))],
        out_specs=pl.BlockSpec((blk, D), lambda i: (i, 0)),
        out_shape=jax.ShapeDtypeStruct((np_, D), F32),
    )(C)


def _tc_update(h, P, inv, u1a, u1b, ub1, u2, ub2, gamma, beta, nb):
    n = h.shape[0]
    blk = n // nb

    def body(h_ref, p_ref, inv_ref, u1a_r, u1b_r, ub1_r, u2_r, ub2_r,
             g_r, bt_r, o_ref):
        agg = (p_ref[0] + p_ref[1]) * inv_ref[...]
        um = jax.nn.silu(
            jnp.dot(h_ref[...], u1a_r[...], preferred_element_type=F32)
            + jnp.dot(agg, u1b_r[...], preferred_element_type=F32)
            + ub1_r[...]
        )
        delta = jnp.dot(um, u2_r[...], preferred_element_type=F32) + ub2_r[...]
        y = h_ref[...] + delta
        mu = jnp.mean(y, axis=-1, keepdims=True)
        var = jnp.mean((y - mu) ** 2, axis=-1, keepdims=True)
        o_ref[...] = (y - mu) / jnp.sqrt(var + 1e-5) * g_r[...] + bt_r[...]

    return pl.pallas_call(
        body,
        grid=(nb,),
        in_specs=[
            pl.BlockSpec((blk, D), lambda i: (i, 0)),
            pl.BlockSpec((NC, blk, D), lambda i: (0, i, 0)),
            pl.BlockSpec((blk, D), lambda i: (i, 0)),
            pl.BlockSpec((D, D), lambda i: (0, 0)),
            pl.BlockSpec((D, D), lambda i: (0, 0)),
            pl.BlockSpec((1, D), lambda i: (0, 0)),
            pl.BlockSpec((D, D), lambda i: (0, 0)),
            pl.BlockSpec((1, D), lambda i: (0, 0)),
            pl.BlockSpec((1, D), lambda i: (0, 0)),
            pl.BlockSpec((1, D), lambda i: (0, 0)),
        ],
        out_specs=pl.BlockSpec((blk, D), lambda i: (i, 0)),
        out_shape=jax.ShapeDtypeStruct((n, D), F32),
    )(h, P, inv, u1a, u1b, ub1, u2, ub2, gamma, beta)


def _pad16(n):
    return ((n + 16 * 128 - 1) // (16 * 128)) * (16 * 128)


def _layer(p, h, gidx, dst, attr, inv, n):
    wa = p['msg_w1'][:D]
    wb = p['msg_w1'][D:2 * D]
    wc = p['msg_w1'][2 * D:]
    wab = jnp.stack([wa, wb])
    bab = jnp.stack([p['msg_b1'], jnp.zeros_like(p['msg_b1'])])[:, None, :]

    AB = _tc_project(h, wab, bab, nb=10).reshape(2 * n, D)
    G = _make_gather(gidx.shape[0], 400)(AB, gidx)
    M = _tc_msg(G, attr, wc, p['msg_w2'], p['msg_b2'][None], eb=2000)
    P = _make_scatter(dst.shape[0], _pad16(n), D, 200)(M, dst)
    return _tc_update(
        h, P, inv,
        p['upd_w1'][:D], p['upd_w1'][D:], p['upd_b1'][None],
        p['upd_w2'], p['upd_b2'][None],
        p['gamma'][None], p['beta'][None], nb=10)


def kernel(h_atom, bond_edge_index, bond_edge_attr,
           radial_edge_index, radial_edge_attr, params):
    n = h_atom.shape[0]
    bsrc = bond_edge_index[0].astype(jnp.int32)
    bdst = bond_edge_index[1].astype(jnp.int32)
    rsrc = radial_edge_index[0].astype(jnp.int32)
    rdst = radial_edge_index[1].astype(jnp.int32)
    bgidx = jnp.concatenate([bsrc, bdst + n])
    rgidx = jnp.concatenate([rsrc, rdst + n])

    bC = _make_scatter(bdst.shape[0], _pad16(n), D, 200, ones=True)(bdst)
    rC = _make_scatter(rdst.shape[0], _pad16(n), D, 200, ones=True)(rdst)
    binv = _tc_invcnt(bC, nb=10)
    rinv = _tc_invcnt(rC, nb=10)

    h = h_atom
    for r in range(2):
        h = _layer(params[2 * r], h, bgidx, bdst, bond_edge_attr, binv, n)
        h = _layer(params[2 * r + 1], h, rgidx, rdst, radial_edge_attr, rinv, n)
    return h


# speedup vs baseline: 1.0026x; 1.0026x over previous
import dataclasses
import functools

import jax
import jax.numpy as jnp
from jax import lax
from jax.experimental import pallas as pl
from jax.experimental.pallas import tpu as pltpu
from jax.experimental.pallas import tpu_sc as plsc

NC = 2
NS = 16
NW = NC * NS
D = 128
F32 = jnp.float32


@functools.lru_cache(maxsize=None)
def _make_gather(rows, k, w=D):
    per = rows // NW
    nch = per // k
    assert per % k == 0 and per % 8 == 0 and k % 8 == 0
    mesh = plsc.VectorSubcoreMesh(core_axis_name="c", subcore_axis_name="s")

    def body(table_hbm, idx_hbm, out_hbm,
             idx0, idx1, buf0, buf1, s0, s1, w0, w1):
        wid = lax.axis_index("s") * NC + lax.axis_index("c")
        base = wid * per
        slots = ((idx0, buf0, s0, w0), (idx1, buf1, s1, w1))

        def load(slot, g):
            ib, rb, sem, _ = slots[slot]
            pltpu.sync_copy(idx_hbm.at[pl.ds(base + g * k, k)], ib)
            pltpu.async_copy(table_hbm.at[ib], rb, sem)

        def drain(slot, g):
            ib, rb, sem, wsem = slots[slot]
            pltpu.make_async_copy(table_hbm.at[ib], rb, sem).wait()
            pltpu.async_copy(rb, out_hbm.at[pl.ds(base + g * k, k)], wsem)

        def waitw(slot, g):
            _, rb, _, wsem = slots[slot]
            pltpu.make_async_copy(
                rb, out_hbm.at[pl.ds(base + g * k, k)], wsem).wait()

        load(0, 0)

        @pl.loop(0, nch, step=2)
        def _(g):
            @pl.when(g + 1 < nch)
            def _():
                @pl.when(g >= 1)
                def _():
                    waitw(1, g - 1)
                load(1, g + 1)
            drain(0, g)

            @pl.when(g + 2 < nch)
            def _():
                waitw(0, g)
                load(0, g + 2)

            @pl.when(g + 1 < nch)
            def _():
                drain(1, g + 1)

        waitw((nch - 1) % 2, nch - 1)
        if nch >= 2:
            waitw((nch - 2) % 2, nch - 2)

    def gather(table, idx):
        return pl.kernel(
            body,
            out_type=jax.ShapeDtypeStruct((rows, w), F32),
            mesh=mesh,
            scratch_types=[
                pltpu.VMEM((k,), jnp.int32),
                pltpu.VMEM((k,), jnp.int32),
                pltpu.VMEM((k, w), F32),
                pltpu.VMEM((k, w), F32),
                pltpu.SemaphoreType.DMA,
                pltpu.SemaphoreType.DMA,
                pltpu.SemaphoreType.DMA,
                pltpu.SemaphoreType.DMA,
            ],
        )(table, idx)

    return gather


@functools.lru_cache(maxsize=None)
def _make_scatter(E, Np, W, k, ones=False):
    per = E // NW
    nch = per // k
    rpt = Np // NS
    zrows = 80
    ka = 96
    kb = k - ka
    assert per % k == 0 and per % 8 == 0 and k % 8 == 0 and rpt % zrows == 0
    assert rpt % 8 == 0 and ka % 8 == 0 and kb % 8 == 0 and zrows <= ka
    mesh = plsc.VectorSubcoreMesh(core_axis_name="c", subcore_axis_name="s")

    def body(*args):
        if ones:
            dst_hbm, out_hbm, acc, idx0, idx1, row0 = args
        else:
            (msgs_hbm, dst_hbm, out_hbm, acc,
             idx0, idx1, row0, row1, s0, s1) = args
        cid = lax.axis_index("c")
        sid = lax.axis_index("s")
        wid = sid * NC + cid
        idxs = (idx0, idx1)

        @pl.loop(0, zrows)
        def _(i):
            for j in range(W // 16):
                row0.at[pl.ds(i, 1), pl.ds(j * 16, 16)][...] = jnp.zeros(
                    (1, 16), F32)

        for j in range(rpt // zrows):
            pltpu.sync_copy(row0.at[pl.ds(0, zrows)],
                            acc.at[pl.ds(sid * rpt + j * zrows, zrows)])
        plsc.subcore_barrier()

        if ones:
            @pl.loop(0, k)
            def _(i):
                for j in range(W // 16):
                    row0.at[pl.ds(i, 1), pl.ds(j * 16, 16)][...] = jnp.ones(
                        (1, 16), F32)

            pltpu.sync_copy(dst_hbm.at[pl.ds(wid * per, k)], idx0)

            @pl.loop(0, nch)
            def _(g):
                pltpu.sync_copy(row0, acc.at[idx0], add=True)

                @pl.when(g + 1 < nch)
                def _():
                    pltpu.sync_copy(
                        dst_hbm.at[pl.ds(wid * per + (g + 1) * k, k)], idx0)
        else:
            rows_ = (row0, row1)
            sems = (s0, s1)
            sizes = (ka, kb)

            def off_of(g, slot):
                return wid * per + g * k + slot * ka

            def load(slot, g):
                pltpu.sync_copy(
                    dst_hbm.at[pl.ds(off_of(g, slot), sizes[slot])],
                    idxs[slot])
                pltpu.async_copy(
                    msgs_hbm.at[pl.ds(off_of(g, slot), sizes[slot])],
                    rows_[slot], sems[slot])

            def addchunk(slot, g):
                pltpu.make_async_copy(
                    msgs_hbm.at[pl.ds(off_of(g, slot), sizes[slot])],
                    rows_[slot], sems[slot]).wait()
                pltpu.sync_copy(rows_[slot], acc.at[idxs[slot]], add=True)

            load(0, 0)

            @pl.loop(0, nch)
            def _(g):
                load(1, g)
                addchunk(0, g)

                @pl.when(g + 1 < nch)
                def _():
                    load(0, g + 1)
                addchunk(1, g)

        plsc.subcore_barrier()
        pltpu.sync_copy(acc.at[pl.ds(sid * rpt, rpt)],
                        out_hbm.at[cid, pl.ds(sid * rpt, rpt)])

    def scatter(*operands):
        if ones:
            scratch = [
                pltpu.VMEM_SHARED((Np, W), F32),
                pltpu.VMEM((k,), jnp.int32),
                pltpu.VMEM((k,), jnp.int32),
                pltpu.VMEM((k, W), F32),
            ]
        else:
            scratch = [
                pltpu.VMEM_SHARED((Np, W), F32),
                pltpu.VMEM((ka,), jnp.int32),
                pltpu.VMEM((kb,), jnp.int32),
                pltpu.VMEM((ka, W), F32),
                pltpu.VMEM((kb, W), F32),
                pltpu.SemaphoreType.DMA,
                pltpu.SemaphoreType.DMA,
            ]
        return pl.kernel(
            body,
            out_type=jax.ShapeDtypeStruct((NC, Np, W), F32),
            mesh=mesh,
            scratch_types=scratch,
        )(*operands)

    return scatter


@functools.lru_cache(maxsize=None)
def _make_count(E, Np):
    per = E // NW
    nfull = per // 16
    rem = per % 16
    assert per % 8 == 0
    mesh = plsc.VectorSubcoreMesh(core_axis_name="c", subcore_axis_name="s")

    def body(dst_hbm, out_hbm, cnt, idxb):
        wid = lax.axis_index("s") * NC + lax.axis_index("c")

        @pl.loop(0, Np // 16)
        def _(i):
            cnt[pl.ds(i * 16, 16)] = jnp.zeros((16,), F32)

        if rem:
            idxb[pl.ds(nfull * 16, 16)] = jnp.zeros((16,), jnp.int32)
        pltpu.sync_copy(dst_hbm.at[pl.ds(wid * per, per)],
                        idxb.at[pl.ds(0, per)])
        ones16 = jnp.ones((16,), F32)

        @pl.loop(0, nfull)
        def _(g):
            iv = idxb[pl.ds(g * 16, 16)]
            plsc.addupdate_scatter(cnt, [iv], ones16)

        if rem:
            iv = idxb[pl.ds(nfull * 16, 16)]
            mask = lax.iota(jnp.int32, 16) < rem
            plsc.addupdate_scatter(cnt, [iv], ones16, mask=mask)

        pltpu.sync_copy(cnt, out_hbm.at[wid])

    cp = pltpu.CompilerParams()
    if "needs_layout_passes" in pltpu.CompilerParams.__dataclass_fields__:
        cp = dataclasses.replace(cp, needs_layout_passes=False)

    def count(dst):
        return pl.kernel(
            body,
            out_type=jax.ShapeDtypeStruct((NW, Np), F32),
            mesh=mesh,
            compiler_params=cp,
            scratch_types=[
                pltpu.VMEM((Np,), F32),
                pltpu.VMEM((nfull * 16 + (16 if rem else 0),), jnp.int32),
            ],
        )(dst)

    return count


def _tc_project(h, wab, bab, nb):
    n = h.shape[0]
    blk = n // nb

    def body(h_ref, w_ref, b_ref, o_ref):
        o_ref[0] = (
            jnp.dot(h_ref[...], w_ref[0], preferred_element_type=F32)
            + b_ref[0]
        )

    return pl.pallas_call(
        body,
        grid=(2, nb),
        in_specs=[
            pl.BlockSpec((blk, D), lambda i, j: (j, 0)),
            pl.BlockSpec((1, D, D), lambda i, j: (i, 0, 0)),
            pl.BlockSpec((1, 1, D), lambda i, j: (i, 0, 0)),
        ],
        out_specs=pl.BlockSpec((1, blk, D), lambda i, j: (i, j, 0)),
        out_shape=jax.ShapeDtypeStruct((2, n, D), F32),
    )(h, wab, bab)


def _tc_msg(G, attr, wc, w2, b2, eb):
    E = G.shape[0] // 2
    da = attr.shape[1]
    nb = E // eb

    def body(ga, gb, at, wc_ref, w2_ref, b2_ref, o_ref):
        x = ga[...] + gb[...] + jnp.dot(
            at[...], wc_ref[...], preferred_element_type=F32)
        hmid = jax.nn.silu(x)
        o_ref[...] = (
            jnp.dot(hmid, w2_ref[...], preferred_element_type=F32)
            + b2_ref[...]
        )

    return pl.pallas_call(
        body,
        grid=(nb,),
        in_specs=[
            pl.BlockSpec((eb, D), lambda i: (i, 0)),
            pl.BlockSpec((eb, D), lambda i, _nb=nb: (i + _nb, 0)),
            pl.BlockSpec((eb, da), lambda i: (i, 0)),
            pl.BlockSpec((da, D), lambda i: (0, 0)),
            pl.BlockSpec((D, D), lambda i: (0, 0)),
            pl.BlockSpec((1, D), lambda i: (0, 0)),
        ],
        out_specs=pl.BlockSpec((eb, D), lambda i: (i, 0)),
        out_shape=jax.ShapeDtypeStruct((E, D), F32),
    )(G, G, attr, wc, w2, b2)


def _tc_invcnt(C, nb):
    np_ = C.shape[1]
    blk = np_ // nb

    def body(c_ref, o_ref):
        s = jnp.sum(c_ref[...], axis=0)
        o_ref[...] = jnp.broadcast_to(
            (1.0 / jnp.maximum(s, 1.0))[:, None], o_ref.shape)

    return pl.pallas_call(
        body,
        grid=(nb,),
        in_specs=[pl.BlockSpec((NW, blk), lambda i: (0, i))],
        out_specs=pl.BlockSpec((blk, D), lambda i: (i, 0)),
        out_shape=jax.ShapeDtypeStruct((np_, D), F32),
    )(C)


def _tc_update(h, P, inv, u1a, u1b, ub1, u2, ub2, gamma, beta, nb):
    n = h.shape[0]
    blk = n // nb

    def body(h_ref, p_ref, inv_ref, u1a_r, u1b_r, ub1_r, u2_r, ub2_r,
             g_r, bt_r, o_ref):
        agg = (p_ref[0] + p_ref[1]) * inv_ref[...]
        um = jax.nn.silu(
            jnp.dot(h_ref[...], u1a_r[...], preferred_element_type=F32)
            + jnp.dot(agg, u1b_r[...], preferred_element_type=F32)
            + ub1_r[...]
        )
        delta = jnp.dot(um, u2_r[...], preferred_element_type=F32) + ub2_r[...]
        y = h_ref[...] + delta
        mu = jnp.mean(y, axis=-1, keepdims=True)
        var = jnp.mean((y - mu) ** 2, axis=-1, keepdims=True)
        o_ref[...] = (y - mu) / jnp.sqrt(var + 1e-5) * g_r[...] + bt_r[...]

    return pl.pallas_call(
        body,
        grid=(nb,),
        in_specs=[
            pl.BlockSpec((blk, D), lambda i: (i, 0)),
            pl.BlockSpec((NC, blk, D), lambda i: (0, i, 0)),
            pl.BlockSpec((blk, D), lambda i: (i, 0)),
            pl.BlockSpec((D, D), lambda i: (0, 0)),
            pl.BlockSpec((D, D), lambda i: (0, 0)),
            pl.BlockSpec((1, D), lambda i: (0, 0)),
            pl.BlockSpec((D, D), lambda i: (0, 0)),
            pl.BlockSpec((1, D), lambda i: (0, 0)),
            pl.BlockSpec((1, D), lambda i: (0, 0)),
            pl.BlockSpec((1, D), lambda i: (0, 0)),
        ],
        out_specs=pl.BlockSpec((blk, D), lambda i: (i, 0)),
        out_shape=jax.ShapeDtypeStruct((n, D), F32),
    )(h, P, inv, u1a, u1b, ub1, u2, ub2, gamma, beta)


def _pad16(n):
    return ((n + 16 * 128 - 1) // (16 * 128)) * (16 * 128)


def _layer(p, h, gidx, dst, attr, inv, n):
    wa = p['msg_w1'][:D]
    wb = p['msg_w1'][D:2 * D]
    wc = p['msg_w1'][2 * D:]
    wab = jnp.stack([wa, wb])
    bab = jnp.stack([p['msg_b1'], jnp.zeros_like(p['msg_b1'])])[:, None, :]

    AB = _tc_project(h, wab, bab, nb=10).reshape(2 * n, D)
    G = _make_gather(gidx.shape[0], 400)(AB, gidx)
    M = _tc_msg(G, attr, wc, p['msg_w2'], p['msg_b2'][None], eb=2000)
    P = _make_scatter(dst.shape[0], _pad16(n), D, 200)(M, dst)
    return _tc_update(
        h, P, inv,
        p['upd_w1'][:D], p['upd_w1'][D:], p['upd_b1'][None],
        p['upd_w2'], p['upd_b2'][None],
        p['gamma'][None], p['beta'][None], nb=10)


def kernel(h_atom, bond_edge_index, bond_edge_attr,
           radial_edge_index, radial_edge_attr, params):
    n = h_atom.shape[0]
    bsrc = bond_edge_index[0].astype(jnp.int32)
    bdst = bond_edge_index[1].astype(jnp.int32)
    rsrc = radial_edge_index[0].astype(jnp.int32)
    rdst = radial_edge_index[1].astype(jnp.int32)
    bgidx = jnp.concatenate([bsrc, bdst + n])
    rgidx = jnp.concatenate([rsrc, rdst + n])

    bC = _make_count(bdst.shape[0], _pad16(n))(bdst)
    rC = _make_count(rdst.shape[0], _pad16(n))(rdst)
    binv = _tc_invcnt(bC, nb=10)
    rinv = _tc_invcnt(rC, nb=10)

    h = h_atom
    for r in range(2):
        h = _layer(params[2 * r], h, bgidx, bdst, bond_edge_attr, binv, n)
        h = _layer(params[2 * r + 1], h, rgidx, rdst, radial_edge_attr, rinv, n)
    return h


# speedup vs baseline: 1.0250x; 1.0223x over previous
import dataclasses
import functools

import jax
import jax.numpy as jnp
from jax import lax
from jax.experimental import pallas as pl
from jax.experimental.pallas import tpu as pltpu
from jax.experimental.pallas import tpu_sc as plsc

NC = 2
NS = 16
NW = NC * NS
D = 128
F32 = jnp.float32


@functools.lru_cache(maxsize=None)
def _make_gather(rows, k, w=D):
    per = rows // NW
    nch = per // k
    assert per % k == 0 and per % 8 == 0 and k % 8 == 0
    mesh = plsc.VectorSubcoreMesh(core_axis_name="c", subcore_axis_name="s")

    def body(table_hbm, idx_hbm, out_hbm,
             idx0, idx1, buf0, buf1, s0, s1, w0, w1):
        wid = lax.axis_index("s") * NC + lax.axis_index("c")
        base = wid * per
        slots = ((idx0, buf0, s0, w0), (idx1, buf1, s1, w1))

        def load(slot, g):
            ib, rb, sem, _ = slots[slot]
            pltpu.sync_copy(idx_hbm.at[pl.ds(base + g * k, k)], ib)
            pltpu.async_copy(table_hbm.at[ib], rb, sem)

        def drain(slot, g):
            ib, rb, sem, wsem = slots[slot]
            pltpu.make_async_copy(table_hbm.at[ib], rb, sem).wait()
            pltpu.async_copy(rb, out_hbm.at[pl.ds(base + g * k, k)], wsem)

        def waitw(slot, g):
            _, rb, _, wsem = slots[slot]
            pltpu.make_async_copy(
                rb, out_hbm.at[pl.ds(base + g * k, k)], wsem).wait()

        load(0, 0)

        @pl.loop(0, nch, step=2)
        def _(g):
            @pl.when(g + 1 < nch)
            def _():
                @pl.when(g >= 1)
                def _():
                    waitw(1, g - 1)
                load(1, g + 1)
            drain(0, g)

            @pl.when(g + 2 < nch)
            def _():
                waitw(0, g)
                load(0, g + 2)

            @pl.when(g + 1 < nch)
            def _():
                drain(1, g + 1)

        waitw((nch - 1) % 2, nch - 1)
        if nch >= 2:
            waitw((nch - 2) % 2, nch - 2)

    def gather(table, idx):
        return pl.kernel(
            body,
            out_type=jax.ShapeDtypeStruct((rows, w), F32),
            mesh=mesh,
            scratch_types=[
                pltpu.VMEM((k,), jnp.int32),
                pltpu.VMEM((k,), jnp.int32),
                pltpu.VMEM((k, w), F32),
                pltpu.VMEM((k, w), F32),
                pltpu.SemaphoreType.DMA,
                pltpu.SemaphoreType.DMA,
                pltpu.SemaphoreType.DMA,
                pltpu.SemaphoreType.DMA,
            ],
        )(table, idx)

    return gather


@functools.lru_cache(maxsize=None)
def _make_scatter(E, Np, W, k, ones=False):
    per = E // NW
    nch = per // k
    rpt = Np // NS
    zrows = 80
    ka = 96
    kb = k - ka
    assert per % k == 0 and per % 8 == 0 and k % 8 == 0 and rpt % zrows == 0
    assert rpt % 8 == 0 and ka % 8 == 0 and kb % 8 == 0 and zrows <= ka
    mesh = plsc.VectorSubcoreMesh(core_axis_name="c", subcore_axis_name="s")

    def body(*args):
        if ones:
            dst_hbm, out_hbm, acc, idx0, idx1, row0 = args
        else:
            (msgs_hbm, dst_hbm, out_hbm, acc,
             idx0, idx1, row0, row1, s0, s1) = args
        cid = lax.axis_index("c")
        sid = lax.axis_index("s")
        wid = sid * NC + cid
        idxs = (idx0, idx1)

        @pl.loop(0, zrows)
        def _(i):
            for j in range(W // 16):
                row0.at[pl.ds(i, 1), pl.ds(j * 16, 16)][...] = jnp.zeros(
                    (1, 16), F32)

        for j in range(rpt // zrows):
            pltpu.sync_copy(row0.at[pl.ds(0, zrows)],
                            acc.at[pl.ds(sid * rpt + j * zrows, zrows)])
        plsc.subcore_barrier()

        if ones:
            @pl.loop(0, k)
            def _(i):
                for j in range(W // 16):
                    row0.at[pl.ds(i, 1), pl.ds(j * 16, 16)][...] = jnp.ones(
                        (1, 16), F32)

            pltpu.sync_copy(dst_hbm.at[pl.ds(wid * per, k)], idx0)

            @pl.loop(0, nch)
            def _(g):
                pltpu.sync_copy(row0, acc.at[idx0], add=True)

                @pl.when(g + 1 < nch)
                def _():
                    pltpu.sync_copy(
                        dst_hbm.at[pl.ds(wid * per + (g + 1) * k, k)], idx0)
        else:
            rows_ = (row0, row1)
            sems = (s0, s1)
            sizes = (ka, kb)

            def off_of(g, slot):
                return wid * per + g * k + slot * ka

            def load(slot, g):
                pltpu.sync_copy(
                    dst_hbm.at[pl.ds(off_of(g, slot), sizes[slot])],
                    idxs[slot])
                pltpu.async_copy(
                    msgs_hbm.at[pl.ds(off_of(g, slot), sizes[slot])],
                    rows_[slot], sems[slot])

            def addchunk(slot, g):
                pltpu.make_async_copy(
                    msgs_hbm.at[pl.ds(off_of(g, slot), sizes[slot])],
                    rows_[slot], sems[slot]).wait()
                pltpu.sync_copy(rows_[slot], acc.at[idxs[slot]], add=True)

            load(0, 0)

            @pl.loop(0, nch)
            def _(g):
                load(1, g)
                addchunk(0, g)

                @pl.when(g + 1 < nch)
                def _():
                    load(0, g + 1)
                addchunk(1, g)

        plsc.subcore_barrier()
        pltpu.sync_copy(acc.at[pl.ds(sid * rpt, rpt)],
                        out_hbm.at[cid, pl.ds(sid * rpt, rpt)])

    def scatter(*operands):
        if ones:
            scratch = [
                pltpu.VMEM_SHARED((Np, W), F32),
                pltpu.VMEM((k,), jnp.int32),
                pltpu.VMEM((k,), jnp.int32),
                pltpu.VMEM((k, W), F32),
            ]
        else:
            scratch = [
                pltpu.VMEM_SHARED((Np, W), F32),
                pltpu.VMEM((ka,), jnp.int32),
                pltpu.VMEM((kb,), jnp.int32),
                pltpu.VMEM((ka, W), F32),
                pltpu.VMEM((kb, W), F32),
                pltpu.SemaphoreType.DMA,
                pltpu.SemaphoreType.DMA,
            ]
        return pl.kernel(
            body,
            out_type=jax.ShapeDtypeStruct((NC, Np, W), F32),
            mesh=mesh,
            scratch_types=scratch,
        )(*operands)

    return scatter


@functools.lru_cache(maxsize=None)
def _make_count(E, Np):
    per = E // NW
    nfull = per // 16
    rem = per % 16
    assert per % 8 == 0
    mesh = plsc.VectorSubcoreMesh(core_axis_name="c", subcore_axis_name="s")

    def body(dst_hbm, out_hbm, cnt, idxb):
        wid = lax.axis_index("s") * NC + lax.axis_index("c")

        @pl.loop(0, Np // 16)
        def _(i):
            cnt[pl.ds(i * 16, 16)] = jnp.zeros((16,), F32)

        if rem:
            idxb[pl.ds(nfull * 16, 16)] = jnp.zeros((16,), jnp.int32)
        pltpu.sync_copy(dst_hbm.at[pl.ds(wid * per, per)],
                        idxb.at[pl.ds(0, per)])
        ones16 = jnp.ones((16,), F32)

        @pl.loop(0, nfull)
        def _(g):
            iv = idxb[pl.ds(g * 16, 16)]
            plsc.addupdate_scatter(cnt, [iv], ones16)

        if rem:
            iv = idxb[pl.ds(nfull * 16, 16)]
            mask = lax.iota(jnp.int32, 16) < rem
            plsc.addupdate_scatter(cnt, [iv], ones16, mask=mask)

        pltpu.sync_copy(cnt, out_hbm.at[wid])

    cp = pltpu.CompilerParams()
    if "needs_layout_passes" in pltpu.CompilerParams.__dataclass_fields__:
        cp = dataclasses.replace(cp, needs_layout_passes=False)

    def count(dst):
        return pl.kernel(
            body,
            out_type=jax.ShapeDtypeStruct((NW, Np), F32),
            mesh=mesh,
            compiler_params=cp,
            scratch_types=[
                pltpu.VMEM((Np,), F32),
                pltpu.VMEM((nfull * 16 + (16 if rem else 0),), jnp.int32),
            ],
        )(dst)

    return count


def _tc_project(h, wab, bab, nb):
    n = h.shape[0]
    blk = n // nb

    def body(h_ref, w_ref, b_ref, o_ref):
        o_ref[0] = (
            jnp.dot(h_ref[...], w_ref[0], preferred_element_type=F32)
            + b_ref[0]
        )

    return pl.pallas_call(
        body,
        grid=(2, nb),
        in_specs=[
            pl.BlockSpec((blk, D), lambda i, j: (j, 0)),
            pl.BlockSpec((1, D, D), lambda i, j: (i, 0, 0)),
            pl.BlockSpec((1, 1, D), lambda i, j: (i, 0, 0)),
        ],
        out_specs=pl.BlockSpec((1, blk, D), lambda i, j: (i, j, 0)),
        out_shape=jax.ShapeDtypeStruct((2, n, D), F32),
    )(h, wab, bab)


def _tc_msg(G, attr, wc, w2, b2, eb):
    E = G.shape[0] // 2
    da = attr.shape[1]
    nb = E // eb

    def body(ga, gb, at, wc_ref, w2_ref, b2_ref, o_ref):
        x = ga[...] + gb[...] + jnp.dot(
            at[...], wc_ref[...], preferred_element_type=F32)
        hmid = jax.nn.silu(x)
        o_ref[...] = (
            jnp.dot(hmid, w2_ref[...], preferred_element_type=F32)
            + b2_ref[...]
        )

    return pl.pallas_call(
        body,
        grid=(nb,),
        in_specs=[
            pl.BlockSpec((eb, D), lambda i: (i, 0)),
            pl.BlockSpec((eb, D), lambda i, _nb=nb: (i + _nb, 0)),
            pl.BlockSpec((eb, da), lambda i: (i, 0)),
            pl.BlockSpec((da, D), lambda i: (0, 0)),
            pl.BlockSpec((D, D), lambda i: (0, 0)),
            pl.BlockSpec((1, D), lambda i: (0, 0)),
        ],
        out_specs=pl.BlockSpec((eb, D), lambda i: (i, 0)),
        out_shape=jax.ShapeDtypeStruct((E, D), F32),
    )(G, G, attr, wc, w2, b2)


def _tc_invcnt(C, nb):
    np_ = C.shape[1]
    blk = np_ // nb

    def body(c_ref, o_ref):
        s = jnp.sum(c_ref[...], axis=0)
        o_ref[...] = jnp.broadcast_to(
            (1.0 / jnp.maximum(s, 1.0))[:, None], o_ref.shape)

    return pl.pallas_call(
        body,
        grid=(nb,),
        in_specs=[pl.BlockSpec((NW, blk), lambda i: (0, i))],
        out_specs=pl.BlockSpec((blk, D), lambda i: (i, 0)),
        out_shape=jax.ShapeDtypeStruct((np_, D), F32),
    )(C)


def _tc_update(h, Ps, inv, u1a, u1b, ub1, u2, ub2, gamma, beta, nb):
    n = h.shape[0]
    blk = n // nb
    ns = len(Ps)

    def body(*refs):
        h_ref = refs[0]
        p_refs = refs[1:1 + ns]
        (inv_ref, u1a_r, u1b_r, ub1_r, u2_r, ub2_r,
         g_r, bt_r, o_ref) = refs[1 + ns:]
        psum = p_refs[0][0] + p_refs[0][1]
        for pr in p_refs[1:]:
            psum = psum + pr[0] + pr[1]
        agg = psum * inv_ref[...]
        um = jax.nn.silu(
            jnp.dot(h_ref[...], u1a_r[...], preferred_element_type=F32)
            + jnp.dot(agg, u1b_r[...], preferred_element_type=F32)
            + ub1_r[...]
        )
        delta = jnp.dot(um, u2_r[...], preferred_element_type=F32) + ub2_r[...]
        y = h_ref[...] + delta
        mu = jnp.mean(y, axis=-1, keepdims=True)
        var = jnp.mean((y - mu) ** 2, axis=-1, keepdims=True)
        o_ref[...] = (y - mu) / jnp.sqrt(var + 1e-5) * g_r[...] + bt_r[...]

    return pl.pallas_call(
        body,
        grid=(nb,),
        in_specs=[
            pl.BlockSpec((blk, D), lambda i: (i, 0)),
        ] + [
            pl.BlockSpec((NC, blk, D), lambda i: (0, i, 0))
            for _ in range(ns)
        ] + [
            pl.BlockSpec((blk, D), lambda i: (i, 0)),
            pl.BlockSpec((D, D), lambda i: (0, 0)),
            pl.BlockSpec((D, D), lambda i: (0, 0)),
            pl.BlockSpec((1, D), lambda i: (0, 0)),
            pl.BlockSpec((D, D), lambda i: (0, 0)),
            pl.BlockSpec((1, D), lambda i: (0, 0)),
            pl.BlockSpec((1, D), lambda i: (0, 0)),
            pl.BlockSpec((1, D), lambda i: (0, 0)),
        ],
        out_specs=pl.BlockSpec((blk, D), lambda i: (i, 0)),
        out_shape=jax.ShapeDtypeStruct((n, D), F32),
    )(h, *Ps, inv, u1a, u1b, ub1, u2, ub2, gamma, beta)


def _pad16(n):
    return ((n + 16 * 128 - 1) // (16 * 128)) * (16 * 128)


def _layer(p, h, slices, inv, n):
    wa = p['msg_w1'][:D]
    wb = p['msg_w1'][D:2 * D]
    wc = p['msg_w1'][2 * D:]
    wab = jnp.stack([wa, wb])
    bab = jnp.stack([p['msg_b1'], jnp.zeros_like(p['msg_b1'])])[:, None, :]

    AB = _tc_project(h, wab, bab, nb=10).reshape(2 * n, D)
    Ps = []
    for gidx_s, dst_s, attr_s in slices:
        G = _make_gather(gidx_s.shape[0], 400)(AB, gidx_s)
        M = _tc_msg(G, attr_s, wc, p['msg_w2'], p['msg_b2'][None], eb=1600)
        Ps.append(_make_scatter(dst_s.shape[0], _pad16(n), D, 200)(M, dst_s))
    return _tc_update(
        h, Ps, inv,
        p['upd_w1'][:D], p['upd_w1'][D:], p['upd_b1'][None],
        p['upd_w2'], p['upd_b2'][None],
        p['gamma'][None], p['beta'][None], nb=10)


def kernel(h_atom, bond_edge_index, bond_edge_attr,
           radial_edge_index, radial_edge_attr, params):
    n = h_atom.shape[0]
    bsrc = bond_edge_index[0].astype(jnp.int32)
    bdst = bond_edge_index[1].astype(jnp.int32)
    rsrc = radial_edge_index[0].astype(jnp.int32)
    rdst = radial_edge_index[1].astype(jnp.int32)

    def make_slices(src, dst, attr, bounds):
        out = []
        for a, b in zip(bounds[:-1], bounds[1:]):
            out.append((jnp.concatenate([src[a:b], dst[a:b] + n]),
                        dst[a:b], attr[a:b]))
        return out

    bslices = make_slices(bsrc, bdst, bond_edge_attr, (0, 76800, 160000))
    rslices = make_slices(rsrc, rdst, radial_edge_attr, (0, 160000, 320000))

    bC = _make_count(bdst.shape[0], _pad16(n))(bdst)
    rC = _make_count(rdst.shape[0], _pad16(n))(rdst)
    binv = _tc_invcnt(bC, nb=10)
    rinv = _tc_invcnt(rC, nb=10)

    h = h_atom
    for r in range(2):
        h = _layer(params[2 * r], h, bslices, binv, n)
        h = _layer(params[2 * r + 1], h, rslices, rinv, n)
    return h
